# initial kernel scaffold (unmeasured)
import functools

import jax
import jax.numpy as jnp
from jax import lax
from jax.experimental import pallas as pl
from jax.experimental.pallas import tpu as pltpu

N_DEV = 4
B = 2
SQ_LOC = 512
SKV = 512
HQ_LOC = 8
DH = 64
D_MODEL = 768
HD_LOC = HQ_LOC * DH


def kernel(x, Wq, K_ext, V_ext, Wo):
    xb = x.astype(jnp.bfloat16)
    wqb = Wq.astype(jnp.bfloat16)
    kb = K_ext.astype(jnp.bfloat16)
    vb = V_ext.astype(jnp.bfloat16)
    wob = Wo.astype(jnp.bfloat16)

    def body(x_ref, wq_ref, k_ref, v_ref, wo_ref, out_ref,
             wq_full, wo_full, send_sems, recv_sems):
        my = lax.axis_index("i")
        left = lax.rem(my + (N_DEV - 1), N_DEV)
        right = lax.rem(my + 1, N_DEV)

        wq_full[pl.ds(my, 1)] = wq_ref[...].reshape(1, D_MODEL, HD_LOC)
        wo_full[pl.ds(my, 1)] = wo_ref[...].reshape(1, HD_LOC, D_MODEL)

        barrier_sem = pltpu.get_barrier_semaphore()
        for nbr in (left, right):
            pl.semaphore_signal(
                barrier_sem, inc=1,
                device_id=(nbr,), device_id_type=pl.DeviceIdType.MESH,
            )
        pl.semaphore_wait(barrier_sem, 2)

        for h in range(N_DEV - 1):
            send_origin = lax.rem(my + (2 * N_DEV - h), N_DEV)
            rdma_q = pltpu.make_async_remote_copy(
                src_ref=wq_full.at[pl.ds(send_origin, 1)],
                dst_ref=wq_full.at[pl.ds(send_origin, 1)],
                send_sem=send_sems.at[0, h],
                recv_sem=recv_sems.at[0, h],
                device_id=(right,),
                device_id_type=pl.DeviceIdType.MESH,
            )
            rdma_o = pltpu.make_async_remote_copy(
                src_ref=wo_full.at[pl.ds(send_origin, 1)],
                dst_ref=wo_full.at[pl.ds(send_origin, 1)],
                send_sem=send_sems.at[1, h],
                recv_sem=recv_sems.at[1, h],
                device_id=(right,),
                device_id_type=pl.DeviceIdType.MESH,
            )
            rdma_q.start()
            rdma_o.start()
            rdma_q.wait()
            rdma_o.wait()

        x2d = x_ref[...].reshape(B * SQ_LOC, D_MODEL)
        q = jnp.concatenate(
            [jnp.dot(x2d, wq_full[c]) for c in range(N_DEV)], axis=1
        )
        qr = q.reshape(B, 2, 4, 64, 32, DH)
        kr = k_ref[...].reshape(B, 2, 4, 64, 32, DH)
        vr = v_ref[...].reshape(B, 2, 4, 64, 32, DH)

        ctx_groups = []
        for g in range(4):
            qg = qr[:, :, g].reshape(B, 128, 32, DH)
            kg = kr[:, :, g].reshape(B, 128, 32, DH)
            vg = vr[:, :, g].reshape(B, 128, 32, DH)
            s = jnp.einsum(
                "bqhd,bkhd->bhqk", qg, kg,
                preferred_element_type=jnp.float32,
            ) * 0.125
            m = jnp.max(s, axis=-1, keepdims=True)
            w = jnp.exp(s - m)
            w = w / jnp.sum(w, axis=-1, keepdims=True)
            ctx_g = jnp.einsum(
                "bhqk,bkhd->bqhd", w.astype(jnp.bfloat16), vg,
            )
            ctx_groups.append(ctx_g.reshape(B, 2, 64, 32, DH))
        ctx = jnp.stack(ctx_groups, axis=2)
        ctx2d = ctx.reshape(B * SQ_LOC, 32 * DH)

        out = jnp.dot(
            ctx2d, wo_full[0], preferred_element_type=jnp.float32
        )
        for c in range(1, N_DEV):
            out = out + jnp.dot(
                ctx2d[:, c * HD_LOC:(c + 1) * HD_LOC], wo_full[c],
                preferred_element_type=jnp.float32,
            )
        out_ref[...] = out.reshape(B, SQ_LOC, D_MODEL)

    return pl.pallas_call(
        body,
        out_shape=jax.ShapeDtypeStruct((B, SQ_LOC, D_MODEL), jnp.float32),
        in_specs=[pl.BlockSpec(memory_space=pltpu.VMEM)] * 5,
        out_specs=pl.BlockSpec(memory_space=pltpu.VMEM),
        scratch_shapes=[
            pltpu.VMEM((N_DEV, D_MODEL, HD_LOC), jnp.bfloat16),
            pltpu.VMEM((N_DEV, HD_LOC, D_MODEL), jnp.bfloat16),
            pltpu.SemaphoreType.DMA((2, N_DEV - 1)),
            pltpu.SemaphoreType.DMA((2, N_DEV - 1)),
        ],
        compiler_params=pltpu.CompilerParams(collective_id=0),
    )(xb, wqb, kb, vb, wob)


# baseline (device time: 124860 ns/iter reference)
import functools

import jax
import jax.numpy as jnp
from jax import lax
from jax.experimental import pallas as pl
from jax.experimental.pallas import tpu as pltpu

N_DEV = 4
B = 2
SQ_LOC = 512
SKV = 512
HQ_LOC = 8
DH = 64
D_MODEL = 768
HD_LOC = HQ_LOC * DH


def kernel(x, Wq, K_ext, V_ext, Wo):
    xb = x.astype(jnp.bfloat16)
    wqb = Wq.astype(jnp.bfloat16)
    kb = K_ext.astype(jnp.bfloat16)
    vb = V_ext.astype(jnp.bfloat16)
    wob = Wo.astype(jnp.bfloat16)

    def body(x_ref, wq_ref, k_ref, v_ref, wo_ref, out_ref,
             wq_full, wo_full, send_sems, recv_sems):
        my = lax.axis_index("i")
        left = lax.rem(my + (N_DEV - 1), N_DEV)
        right = lax.rem(my + 1, N_DEV)

        wq_full[pl.ds(my, 1)] = wq_ref[...].reshape(1, D_MODEL, HD_LOC)
        wo_full[pl.ds(my, 1)] = wo_ref[...].reshape(1, HD_LOC, D_MODEL)

        barrier_sem = pltpu.get_barrier_semaphore()
        for nbr in (left, right):
            pl.semaphore_signal(
                barrier_sem, inc=1,
                device_id=(nbr,), device_id_type=pl.DeviceIdType.MESH,
            )
        pl.semaphore_wait(barrier_sem, 2)

        for h in range(N_DEV - 1):
            send_origin = lax.rem(my + (2 * N_DEV - h), N_DEV)
            rdma_q = pltpu.make_async_remote_copy(
                src_ref=wq_full.at[pl.ds(send_origin, 1)],
                dst_ref=wq_full.at[pl.ds(send_origin, 1)],
                send_sem=send_sems.at[0, h],
                recv_sem=recv_sems.at[0, h],
                device_id=(right,),
                device_id_type=pl.DeviceIdType.MESH,
            )
            rdma_o = pltpu.make_async_remote_copy(
                src_ref=wo_full.at[pl.ds(send_origin, 1)],
                dst_ref=wo_full.at[pl.ds(send_origin, 1)],
                send_sem=send_sems.at[1, h],
                recv_sem=recv_sems.at[1, h],
                device_id=(right,),
                device_id_type=pl.DeviceIdType.MESH,
            )
            rdma_q.start()
            rdma_o.start()
            rdma_q.wait()
            rdma_o.wait()

        x2d = x_ref[...].reshape(B * SQ_LOC, D_MODEL)
        q = jnp.concatenate(
            [
                jnp.dot(
                    x2d, wq_full[c], preferred_element_type=jnp.float32
                ).astype(jnp.bfloat16)
                for c in range(N_DEV)
            ],
            axis=1,
        )
        qr = q.reshape(B, 2, 4, 64, 32, DH)
        kr = k_ref[...].reshape(B, 2, 4, 64, 32, DH)
        vr = v_ref[...].reshape(B, 2, 4, 64, 32, DH)

        ctx_groups = []
        for g in range(4):
            qg = (
                qr[:, :, g].reshape(B, 128, 32, DH)
                .transpose(0, 2, 1, 3).reshape(B * 32, 128, DH)
            )
            kg = (
                kr[:, :, g].reshape(B, 128, 32, DH)
                .transpose(0, 2, 1, 3).reshape(B * 32, 128, DH)
            )
            vg = (
                vr[:, :, g].reshape(B, 128, 32, DH)
                .transpose(0, 2, 1, 3).reshape(B * 32, 128, DH)
            )
            s = lax.dot_general(
                qg, kg, (((2,), (2,)), ((0,), (0,))),
                preferred_element_type=jnp.float32,
            ) * 0.125
            m = jnp.max(s, axis=-1, keepdims=True)
            w = jnp.exp(s - m)
            w = w / jnp.sum(w, axis=-1, keepdims=True)
            ctx_g = lax.dot_general(
                w.astype(jnp.bfloat16), vg, (((2,), (1,)), ((0,), (0,))),
                preferred_element_type=jnp.float32,
            ).astype(jnp.bfloat16)
            ctx_g = (
                ctx_g.reshape(B, 32, 128, DH).transpose(0, 2, 1, 3)
            )
            ctx_groups.append(ctx_g.reshape(B, 2, 64, 32, DH))
        ctx = jnp.stack(ctx_groups, axis=2)
        ctx2d = ctx.reshape(B * SQ_LOC, 32 * DH)

        out = jnp.zeros((B * SQ_LOC, D_MODEL), jnp.float32)
        for c in range(N_DEV):
            out = out + jnp.dot(
                ctx2d[:, c * HD_LOC:(c + 1) * HD_LOC], wo_full[c],
                preferred_element_type=jnp.float32,
            )
        out_ref[...] = out.reshape(B, SQ_LOC, D_MODEL)

    return pl.pallas_call(
        body,
        out_shape=jax.ShapeDtypeStruct((B, SQ_LOC, D_MODEL), jnp.float32),
        in_specs=[pl.BlockSpec(memory_space=pltpu.VMEM)] * 5,
        out_specs=pl.BlockSpec(memory_space=pltpu.VMEM),
        scratch_shapes=[
            pltpu.VMEM((N_DEV, D_MODEL, HD_LOC), jnp.bfloat16),
            pltpu.VMEM((N_DEV, HD_LOC, D_MODEL), jnp.bfloat16),
            pltpu.SemaphoreType.DMA((2, N_DEV - 1)),
            pltpu.SemaphoreType.DMA((2, N_DEV - 1)),
        ],
        compiler_params=pltpu.CompilerParams(collective_id=0),
    )(xb, wqb, kb, vb, wob)


# device time: 106206 ns/iter; 1.1756x vs baseline; 1.1756x over previous
import jax
import jax.numpy as jnp
from jax import lax
from jax.experimental import pallas as pl
from jax.experimental.pallas import tpu as pltpu

N_DEV = 4
B = 2
SQ_LOC = 512
SKV = 512
DH = 64
D_MODEL = 768
HQ = 32
HD_LOC = 8 * DH


def kernel(x, Wq, K_ext, V_ext, Wo):
    xb = x.astype(jnp.bfloat16)
    wqb = Wq.astype(jnp.bfloat16)
    kt = K_ext.transpose(0, 2, 1, 3).astype(jnp.bfloat16)
    vt = V_ext.transpose(0, 2, 1, 3).astype(jnp.bfloat16)
    wob = Wo.astype(jnp.bfloat16)

    def body(x_ref, wq_ref, k_ref, v_ref, wo_ref, out_ref,
             wq_full, wo_full, send_sems, recv_sems):
        my = lax.axis_index("i")
        left = lax.rem(my + (N_DEV - 1), N_DEV)
        right = lax.rem(my + 1, N_DEV)

        wq_full[:, pl.ds(my * HD_LOC, HD_LOC)] = wq_ref[...]
        wo_full[pl.ds(my * HD_LOC, HD_LOC), :] = wo_ref[...]

        barrier_sem = pltpu.get_barrier_semaphore()
        for nbr in (left, right):
            pl.semaphore_signal(
                barrier_sem, inc=1,
                device_id=(nbr,), device_id_type=pl.DeviceIdType.MESH,
            )
        pl.semaphore_wait(barrier_sem, 2)

        for h in range(N_DEV - 1):
            send_origin = lax.rem(my + (2 * N_DEV - h), N_DEV)
            col = send_origin * HD_LOC
            rdma_q = pltpu.make_async_remote_copy(
                src_ref=wq_full.at[:, pl.ds(col, HD_LOC)],
                dst_ref=wq_full.at[:, pl.ds(col, HD_LOC)],
                send_sem=send_sems.at[0, h],
                recv_sem=recv_sems.at[0, h],
                device_id=(right,),
                device_id_type=pl.DeviceIdType.MESH,
            )
            rdma_o = pltpu.make_async_remote_copy(
                src_ref=wo_full.at[pl.ds(col, HD_LOC), :],
                dst_ref=wo_full.at[pl.ds(col, HD_LOC), :],
                send_sem=send_sems.at[1, h],
                recv_sem=recv_sems.at[1, h],
                device_id=(right,),
                device_id_type=pl.DeviceIdType.MESH,
            )
            rdma_q.start()
            rdma_o.start()
            rdma_q.wait()
            rdma_o.wait()

        x2d = x_ref[...].reshape(B * SQ_LOC, D_MODEL)
        q = jnp.dot(
            x2d, wq_full[...], preferred_element_type=jnp.float32
        ).astype(jnp.bfloat16)

        qt = (
            q.reshape(B, SQ_LOC, HQ, DH)
            .transpose(0, 2, 1, 3)
            .reshape(B * HQ, 2, 4, 64, DH)
        )
        kr = k_ref[...].reshape(B * HQ, 2, 4, 64, DH)
        vr = v_ref[...].reshape(B * HQ, 2, 4, 64, DH)

        ctx_groups = []
        for g in range(4):
            qg = qt[:, :, g].reshape(B * HQ, 128, DH)
            kg = kr[:, :, g].reshape(B * HQ, 128, DH)
            vg = vr[:, :, g].reshape(B * HQ, 128, DH)
            s = lax.dot_general(
                qg, kg, (((2,), (2,)), ((0,), (0,))),
                preferred_element_type=jnp.float32,
            ) * 0.125
            m = jnp.max(s, axis=-1, keepdims=True)
            w = jnp.exp(s - m)
            w = w / jnp.sum(w, axis=-1, keepdims=True)
            ctx_g = lax.dot_general(
                w.astype(jnp.bfloat16), vg, (((2,), (1,)), ((0,), (0,))),
                preferred_element_type=jnp.float32,
            ).astype(jnp.bfloat16)
            ctx_groups.append(ctx_g.reshape(B * HQ, 2, 64, DH))
        ctx = (
            jnp.stack(ctx_groups, axis=2)
            .reshape(B, HQ, SQ_LOC, DH)
            .transpose(0, 2, 1, 3)
            .reshape(B * SQ_LOC, HQ * DH)
        )
        out = jnp.dot(ctx, wo_full[...], preferred_element_type=jnp.float32)
        out_ref[...] = out.reshape(B, SQ_LOC, D_MODEL)

    return pl.pallas_call(
        body,
        out_shape=jax.ShapeDtypeStruct((B, SQ_LOC, D_MODEL), jnp.float32),
        in_specs=[pl.BlockSpec(memory_space=pltpu.VMEM)] * 5,
        out_specs=pl.BlockSpec(memory_space=pltpu.VMEM),
        scratch_shapes=[
            pltpu.VMEM((D_MODEL, N_DEV * HD_LOC), jnp.bfloat16),
            pltpu.VMEM((N_DEV * HD_LOC, D_MODEL), jnp.bfloat16),
            pltpu.SemaphoreType.DMA((2, N_DEV - 1)),
            pltpu.SemaphoreType.DMA((2, N_DEV - 1)),
        ],
        compiler_params=pltpu.CompilerParams(collective_id=0),
    )(xb, wqb, kt, vt, wob)


# device time: 81142 ns/iter; 1.5388x vs baseline; 1.3089x over previous
import os

import jax
import jax.numpy as jnp
from jax import lax
from jax.experimental import pallas as pl
from jax.experimental.pallas import tpu as pltpu

_DIAG = os.environ.get("KERNEL_DIAG", "")

N_DEV = 4
B = 2
SQ_LOC = 512
SKV = 512
DH = 64
D_MODEL = 768
HQ = 32
HD_LOC = 8 * DH


def kernel(x, Wq, K_ext, V_ext, Wo):
    xb = x.astype(jnp.bfloat16)
    wqb = Wq.astype(jnp.bfloat16)
    kt = K_ext.transpose(0, 2, 1, 3).astype(jnp.bfloat16)
    vt = V_ext.transpose(0, 2, 1, 3).astype(jnp.bfloat16)
    wob = Wo.astype(jnp.bfloat16)

    def body(x_ref, wq_ref, k_ref, v_ref, wo_ref, out_ref,
             wq_full, wo_full, send_sems, recv_sems):
        my = lax.axis_index("i")
        left = lax.rem(my + (N_DEV - 1), N_DEV)
        right = lax.rem(my + 1, N_DEV)

        wq_full[:, pl.ds(my * HD_LOC, HD_LOC)] = wq_ref[...]
        wo_full[pl.ds(my * HD_LOC, HD_LOC), :] = wo_ref[...]

        if _DIAG == "compute_only":
            for c in range(N_DEV):
                wq_full[:, c * HD_LOC:(c + 1) * HD_LOC] = wq_ref[...]
                wo_full[c * HD_LOC:(c + 1) * HD_LOC, :] = wo_ref[...]

        barrier_sem = pltpu.get_barrier_semaphore()
        for nbr in (left, right):
            pl.semaphore_signal(
                barrier_sem, inc=1,
                device_id=(nbr,), device_id_type=pl.DeviceIdType.MESH,
            )
        pl.semaphore_wait(barrier_sem, 2)

        for h in range(N_DEV - 1) if _DIAG != "compute_only" else []:
            send_origin = lax.rem(my + (2 * N_DEV - h), N_DEV)
            col = send_origin * HD_LOC
            rdma_q = pltpu.make_async_remote_copy(
                src_ref=wq_full.at[:, pl.ds(col, HD_LOC)],
                dst_ref=wq_full.at[:, pl.ds(col, HD_LOC)],
                send_sem=send_sems.at[0, h],
                recv_sem=recv_sems.at[0, h],
                device_id=(right,),
                device_id_type=pl.DeviceIdType.MESH,
            )
            rdma_o = pltpu.make_async_remote_copy(
                src_ref=wo_full.at[pl.ds(col, HD_LOC), :],
                dst_ref=wo_full.at[pl.ds(col, HD_LOC), :],
                send_sem=send_sems.at[1, h],
                recv_sem=recv_sems.at[1, h],
                device_id=(right,),
                device_id_type=pl.DeviceIdType.MESH,
            )
            rdma_q.start()
            rdma_o.start()
            rdma_q.wait()
            rdma_o.wait()

        if _DIAG == "comm_only":
            out_ref[...] = jnp.zeros((B, SQ_LOC, D_MODEL), jnp.float32)
            return

        x2d = x_ref[...].reshape(B * SQ_LOC, D_MODEL)
        q = jnp.dot(
            x2d, wq_full[...], preferred_element_type=jnp.float32
        ).astype(jnp.bfloat16)

        qt = (
            q.reshape(B, SQ_LOC, HQ, DH)
            .transpose(0, 2, 1, 3)
            .reshape(B * HQ, 2, 4, 64, DH)
        )
        kr = k_ref[...].reshape(B * HQ, 2, 4, 64, DH)
        vr = v_ref[...].reshape(B * HQ, 2, 4, 64, DH)

        ctx_groups = []
        for g in range(4):
            qg = qt[:, :, g].reshape(B * HQ, 128, DH)
            kg = kr[:, :, g].reshape(B * HQ, 128, DH)
            vg = vr[:, :, g].reshape(B * HQ, 128, DH)
            s = lax.dot_general(
                qg, kg, (((2,), (2,)), ((0,), (0,))),
                preferred_element_type=jnp.float32,
            ) * 0.125
            m = jnp.max(s, axis=-1, keepdims=True)
            w = jnp.exp(s - m)
            w = w / jnp.sum(w, axis=-1, keepdims=True)
            ctx_g = lax.dot_general(
                w.astype(jnp.bfloat16), vg, (((2,), (1,)), ((0,), (0,))),
                preferred_element_type=jnp.float32,
            ).astype(jnp.bfloat16)
            ctx_groups.append(ctx_g.reshape(B * HQ, 2, 64, DH))
        ctx = (
            jnp.stack(ctx_groups, axis=2)
            .reshape(B, HQ, SQ_LOC, DH)
            .transpose(0, 2, 1, 3)
            .reshape(B * SQ_LOC, HQ * DH)
        )
        out = jnp.dot(ctx, wo_full[...], preferred_element_type=jnp.float32)
        out_ref[...] = out.reshape(B, SQ_LOC, D_MODEL)

    return pl.pallas_call(
        body,
        out_shape=jax.ShapeDtypeStruct((B, SQ_LOC, D_MODEL), jnp.float32),
        in_specs=[pl.BlockSpec(memory_space=pltpu.VMEM)] * 5,
        out_specs=pl.BlockSpec(memory_space=pltpu.VMEM),
        scratch_shapes=[
            pltpu.VMEM((D_MODEL, N_DEV * HD_LOC), jnp.bfloat16),
            pltpu.VMEM((N_DEV * HD_LOC, D_MODEL), jnp.bfloat16),
            pltpu.SemaphoreType.DMA((2, N_DEV - 1)),
            pltpu.SemaphoreType.DMA((2, N_DEV - 1)),
        ],
        compiler_params=pltpu.CompilerParams(collective_id=0),
    )(xb, wqb, kt, vt, wob)


# device time: 41953 ns/iter; 2.9762x vs baseline; 1.9341x over previous
import os

import jax
import jax.numpy as jnp
from jax import lax
from jax.experimental import pallas as pl
from jax.experimental.pallas import tpu as pltpu

_DIAG = os.environ.get("KERNEL_DIAG", "")

N_DEV = 4
B = 2
SQ_LOC = 512
SKV = 512
DH = 64
D_MODEL = 768
HQ = 32
H_LOC = 8
HD_LOC = H_LOC * DH


def kernel(x, Wq, K_ext, V_ext, Wo):
    xb = x.astype(jnp.bfloat16)
    wqb = Wq.astype(jnp.bfloat16)
    kt = K_ext.transpose(2, 0, 1, 3).astype(jnp.bfloat16)
    vt = V_ext.transpose(2, 0, 1, 3).astype(jnp.bfloat16)
    wob = Wo.astype(jnp.bfloat16)

    def chunk_contrib(x2d, k_ref, v_ref, wq_c, wo_c, c):
        q_c = jnp.dot(x2d, wq_c, preferred_element_type=jnp.float32)
        q_c = (q_c * 0.125).astype(jnp.bfloat16)
        qhb = (
            q_c.reshape(B, SQ_LOC, H_LOC, DH)
            .transpose(2, 0, 1, 3)
            .reshape(H_LOC * B, 2, 4, 64, DH)
        )
        kc = k_ref[pl.ds(c * H_LOC, H_LOC)].reshape(H_LOC * B, 2, 4, 64, DH)
        vc = v_ref[pl.ds(c * H_LOC, H_LOC)].reshape(H_LOC * B, 2, 4, 64, DH)
        ctx_groups = []
        for g in range(4):
            qg = qhb[:, :, g].reshape(H_LOC * B, 128, DH)
            kg = kc[:, :, g].reshape(H_LOC * B, 128, DH)
            vg = vc[:, :, g].reshape(H_LOC * B, 128, DH)
            s = lax.dot_general(
                qg, kg, (((2,), (2,)), ((0,), (0,))),
                preferred_element_type=jnp.float32,
            )
            w = jnp.exp(s)
            r = 1.0 / jnp.sum(w, axis=-1, keepdims=True)
            ctx_g = lax.dot_general(
                w.astype(jnp.bfloat16), vg, (((2,), (1,)), ((0,), (0,))),
                preferred_element_type=jnp.float32,
            )
            ctx_g = (ctx_g * r).astype(jnp.bfloat16)
            ctx_groups.append(ctx_g.reshape(H_LOC * B, 2, 64, DH))
        ctx = (
            jnp.stack(ctx_groups, axis=2)
            .reshape(H_LOC, B, SQ_LOC, DH)
            .transpose(1, 2, 0, 3)
            .reshape(B * SQ_LOC, HD_LOC)
        )
        return jnp.dot(ctx, wo_c, preferred_element_type=jnp.float32)

    def body(x_ref, wq_ref, k_ref, v_ref, wo_ref, out_ref,
             wq_full, wo_full, send_sems, recv_sems):
        my = lax.axis_index("i")
        left = lax.rem(my + (N_DEV - 1), N_DEV)
        right = lax.rem(my + 1, N_DEV)
        c_left = left
        c_right = right
        c_diag = lax.rem(my + 2, N_DEV)

        wq_full[pl.ds(my, 1)] = wq_ref[...].reshape(1, D_MODEL, HD_LOC)
        wo_full[pl.ds(my, 1)] = wo_ref[...].reshape(1, HD_LOC, D_MODEL)

        x2d = x_ref[...].reshape(B * SQ_LOC, D_MODEL)

        barrier_sem = pltpu.get_barrier_semaphore()
        for nbr in (left, right):
            pl.semaphore_signal(
                barrier_sem, inc=1,
                device_id=(nbr,), device_id_type=pl.DeviceIdType.MESH,
            )
        pl.semaphore_wait(barrier_sem, 2)

        if _DIAG == "compute_only":
            out = chunk_contrib(x2d, k_ref, v_ref, wq_ref[...], wo_ref[...], my)
            for t in range(1, N_DEV):
                c = lax.rem(my + t, N_DEV)
                out = out + chunk_contrib(
                    x2d, k_ref, v_ref, wq_ref[...], wo_ref[...], c
                )
            out_ref[...] = out.reshape(B, SQ_LOC, D_MODEL)
            return

        def rcopy(src, dst, si, ri, dev):
            return pltpu.make_async_remote_copy(
                src_ref=src, dst_ref=dst,
                send_sem=send_sems.at[si], recv_sem=recv_sems.at[ri],
                device_id=(dev,), device_id_type=pl.DeviceIdType.MESH,
            )

        s1 = [
            rcopy(wq_full.at[pl.ds(my, 1)], wq_full.at[pl.ds(my, 1)], 0, 0, right),
            rcopy(wo_full.at[pl.ds(my, 1)], wo_full.at[pl.ds(my, 1)], 1, 1, right),
            rcopy(wq_full.at[pl.ds(my, 1)], wq_full.at[pl.ds(my, 1)], 2, 2, left),
            rcopy(wo_full.at[pl.ds(my, 1)], wo_full.at[pl.ds(my, 1)], 3, 3, left),
        ]
        for r in s1:
            r.start()

        out = chunk_contrib(x2d, k_ref, v_ref, wq_full[my], wo_full[my], my)

        HQH, HOH = D_MODEL // 2, HD_LOC // 2
        s1[0].wait_recv()
        s1[1].wait_recv()
        fwd_r = [
            rcopy(wq_full.at[pl.ds(c_left, 1), pl.ds(0, HQH)],
                  wq_full.at[pl.ds(c_left, 1), pl.ds(0, HQH)], 4, 4, right),
            rcopy(wo_full.at[pl.ds(c_left, 1), pl.ds(0, HOH)],
                  wo_full.at[pl.ds(c_left, 1), pl.ds(0, HOH)], 5, 5, right),
        ]
        for r in fwd_r:
            r.start()
        s1[2].wait_recv()
        s1[3].wait_recv()
        fwd_l = [
            rcopy(wq_full.at[pl.ds(c_right, 1), pl.ds(HQH, HQH)],
                  wq_full.at[pl.ds(c_right, 1), pl.ds(HQH, HQH)], 6, 6, left),
            rcopy(wo_full.at[pl.ds(c_right, 1), pl.ds(HOH, HOH)],
                  wo_full.at[pl.ds(c_right, 1), pl.ds(HOH, HOH)], 7, 7, left),
        ]
        for r in fwd_l:
            r.start()

        out = out + chunk_contrib(
            x2d, k_ref, v_ref, wq_full[c_left], wo_full[c_left], c_left
        )
        out = out + chunk_contrib(
            x2d, k_ref, v_ref, wq_full[c_right], wo_full[c_right], c_right
        )

        for r in fwd_r + fwd_l:
            r.wait_recv()
        out = out + chunk_contrib(
            x2d, k_ref, v_ref, wq_full[c_diag], wo_full[c_diag], c_diag
        )
        out_ref[...] = out.reshape(B, SQ_LOC, D_MODEL)

        for r in s1 + fwd_r + fwd_l:
            r.wait_send()

    return pl.pallas_call(
        body,
        out_shape=jax.ShapeDtypeStruct((B, SQ_LOC, D_MODEL), jnp.float32),
        in_specs=[pl.BlockSpec(memory_space=pltpu.VMEM)] * 5,
        out_specs=pl.BlockSpec(memory_space=pltpu.VMEM),
        scratch_shapes=[
            pltpu.VMEM((N_DEV, D_MODEL, HD_LOC), jnp.bfloat16),
            pltpu.VMEM((N_DEV, HD_LOC, D_MODEL), jnp.bfloat16),
            pltpu.SemaphoreType.DMA((8,)),
            pltpu.SemaphoreType.DMA((8,)),
        ],
        compiler_params=pltpu.CompilerParams(collective_id=0),
    )(xb, wqb, kt, vt, wob)
